# baseline (speedup 1.0000x reference)
import jax
import jax.numpy as jnp
import numpy as np
from jax.experimental import pallas as pl

_B, _N, _K, _D = 8, 2048, 16, 128
_TILE = 256


def _knn_body(lxc, lyc, lxr, lyr, ew, bt, idx_out, attr_out):
    b = pl.program_id(0)
    i = pl.program_id(1)

    xc = lxc[0]
    yc = lyc[0]
    xr = lxr[0]
    yr = lyr[0]

    xcb = xc.astype(jnp.bfloat16).astype(jnp.float32)
    ycb = yc.astype(jnp.bfloat16).astype(jnp.float32)
    xrb = xr.astype(jnp.bfloat16).astype(jnp.float32)
    yrb = yr.astype(jnp.bfloat16).astype(jnp.float32)
    dot = xcb * xrb + ycb * yrb
    sqc = xc * xc + yc * yc
    sqr = xr * xr + yr * yr
    d2 = (sqc + sqr) - 2.0 * dot

    row_g = jax.lax.broadcasted_iota(jnp.int32, (_TILE, _N), 0) + i * _TILE
    col = jax.lax.broadcasted_iota(jnp.int32, (_TILE, _N), 1)
    vals = jnp.where(row_g == col, jnp.inf, jnp.maximum(d2, 1e-12))

    key = (jax.lax.bitcast_convert_type(vals, jnp.int32)
           & jnp.int32(~0x7FF)) | col

    koi = jax.lax.broadcasted_iota(jnp.int32, (_TILE, _K), 1)
    ksel = jnp.zeros((_TILE, _K), jnp.int32)

    for j in range(_K):
        m = jnp.min(key, axis=1, keepdims=True)
        key = jnp.where(key == m, jnp.int32(0x7FFFFFFF), key)
        ksel = jnp.where(koi == j, m, ksel)

    isel = ksel & jnp.int32(0x7FF)
    dsel = jax.lax.bitcast_convert_type(ksel & jnp.int32(~0x7FF), jnp.float32)

    idx_out[0] = isel + b * _N
    ed = jnp.sqrt(dsel)
    attr = jax.lax.dot_general(
        ed, ew[...],
        dimension_numbers=(((1,), (0,)), ((), ())),
        preferred_element_type=jnp.float32,
        precision=jax.lax.Precision.HIGHEST,
    )
    attr_out[0] = attr + bt[...]


def kernel(locs, init_embeddings, W, b):
    Bv, Nv, _ = locs.shape
    lxc = locs[:, :, 0:1]
    lyc = locs[:, :, 1:2]
    lxr = locs[:, :, 0].reshape(Bv, 1, Nv)
    lyr = locs[:, :, 1].reshape(Bv, 1, Nv)
    Wv = W.reshape(_D)
    ew = jnp.kron(jnp.eye(_K, dtype=jnp.float32), Wv[None, :])
    bt = jnp.tile(b, _K)[None, :]

    grid = (Bv, Nv // _TILE)
    idx_out, attr_out = pl.pallas_call(
        _knn_body,
        grid=grid,
        in_specs=[
            pl.BlockSpec((1, _TILE, 1), lambda b_, i: (b_, i, 0)),
            pl.BlockSpec((1, _TILE, 1), lambda b_, i: (b_, i, 0)),
            pl.BlockSpec((1, 1, Nv), lambda b_, i: (b_, 0, 0)),
            pl.BlockSpec((1, 1, Nv), lambda b_, i: (b_, 0, 0)),
            pl.BlockSpec((_K, _K * _D), lambda b_, i: (0, 0)),
            pl.BlockSpec((1, _K * _D), lambda b_, i: (0, 0)),
        ],
        out_specs=[
            pl.BlockSpec((1, _TILE, _K), lambda b_, i: (b_, i, 0)),
            pl.BlockSpec((1, _TILE, _K * _D), lambda b_, i: (b_, i, 0)),
        ],
        out_shape=[
            jax.ShapeDtypeStruct((Bv, Nv, _K), jnp.int32),
            jax.ShapeDtypeStruct((Bv, Nv, _K * _D), jnp.float32),
        ],
    )(lxc, lyc, lxr, lyr, ew, bt)

    x = init_embeddings.reshape(Bv * Nv, _D)
    src = jnp.broadcast_to(
        jnp.arange(Bv * Nv, dtype=jnp.int32)[:, None], (Bv * Nv, _K)
    ).reshape(-1)
    dst = idx_out.reshape(-1)
    edge_index = jnp.stack([src, dst], axis=0)
    edge_attr = attr_out.reshape(Bv * Nv * _K, _D)
    return x, edge_index, edge_attr


# speedup vs baseline: 1.0276x; 1.0276x over previous
import jax
import jax.numpy as jnp
import numpy as np
from jax.experimental import pallas as pl

_B, _N, _K, _D = 8, 2048, 16, 128
_TILE = 512


def _knn_body(lxc, lyc, lxr, lyr, ew, bt, idx_out, attr_out):
    b = pl.program_id(0)
    i = pl.program_id(1)

    xc = lxc[0]
    yc = lyc[0]
    xr = lxr[0]
    yr = lyr[0]

    xcb = xc.astype(jnp.bfloat16).astype(jnp.float32)
    ycb = yc.astype(jnp.bfloat16).astype(jnp.float32)
    xrb = xr.astype(jnp.bfloat16).astype(jnp.float32)
    yrb = yr.astype(jnp.bfloat16).astype(jnp.float32)
    dot = xcb * xrb + ycb * yrb
    sqc = xc * xc + yc * yc
    sqr = xr * xr + yr * yr
    d2 = (sqc + sqr) - 2.0 * dot

    row_g = jax.lax.broadcasted_iota(jnp.int32, (_TILE, _N), 0) + i * _TILE
    col = jax.lax.broadcasted_iota(jnp.int32, (_TILE, _N), 1)
    vals = jnp.where(row_g == col, jnp.inf, jnp.maximum(d2, 1e-12))

    key = (jax.lax.bitcast_convert_type(vals, jnp.int32)
           & jnp.int32(~0x7FF)) | col

    koi = jax.lax.broadcasted_iota(jnp.int32, (_TILE, _K), 1)
    ksel = jnp.zeros((_TILE, _K), jnp.int32)

    for j in range(_K):
        m = jnp.min(key, axis=1, keepdims=True)
        key = jnp.where(key == m, jnp.int32(0x7FFFFFFF), key)
        ksel = jnp.where(koi == j, m, ksel)

    isel = ksel & jnp.int32(0x7FF)
    dsel = jax.lax.bitcast_convert_type(ksel & jnp.int32(~0x7FF), jnp.float32)

    idx_out[0] = isel + b * _N
    ed = jnp.sqrt(dsel)
    attr = jax.lax.dot_general(
        ed, ew[...],
        dimension_numbers=(((1,), (0,)), ((), ())),
        preferred_element_type=jnp.float32,
        precision=jax.lax.Precision.HIGHEST,
    )
    attr_out[0] = attr + bt[...]


def kernel(locs, init_embeddings, W, b):
    Bv, Nv, _ = locs.shape
    lxc = locs[:, :, 0:1]
    lyc = locs[:, :, 1:2]
    lxr = locs[:, :, 0].reshape(Bv, 1, Nv)
    lyr = locs[:, :, 1].reshape(Bv, 1, Nv)
    Wv = W.reshape(_D)
    ew = jnp.kron(jnp.eye(_K, dtype=jnp.float32), Wv[None, :])
    bt = jnp.tile(b, _K)[None, :]

    grid = (Bv, Nv // _TILE)
    idx_out, attr_out = pl.pallas_call(
        _knn_body,
        grid=grid,
        in_specs=[
            pl.BlockSpec((1, _TILE, 1), lambda b_, i: (b_, i, 0)),
            pl.BlockSpec((1, _TILE, 1), lambda b_, i: (b_, i, 0)),
            pl.BlockSpec((1, 1, Nv), lambda b_, i: (b_, 0, 0)),
            pl.BlockSpec((1, 1, Nv), lambda b_, i: (b_, 0, 0)),
            pl.BlockSpec((_K, _K * _D), lambda b_, i: (0, 0)),
            pl.BlockSpec((1, _K * _D), lambda b_, i: (0, 0)),
        ],
        out_specs=[
            pl.BlockSpec((1, _TILE, _K), lambda b_, i: (b_, i, 0)),
            pl.BlockSpec((1, _TILE, _K * _D), lambda b_, i: (b_, i, 0)),
        ],
        out_shape=[
            jax.ShapeDtypeStruct((Bv, Nv, _K), jnp.int32),
            jax.ShapeDtypeStruct((Bv, Nv, _K * _D), jnp.float32),
        ],
    )(lxc, lyc, lxr, lyr, ew, bt)

    x = init_embeddings.reshape(Bv * Nv, _D)
    src = jnp.broadcast_to(
        jnp.arange(Bv * Nv, dtype=jnp.int32)[:, None], (Bv * Nv, _K)
    ).reshape(-1)
    dst = idx_out.reshape(-1)
    edge_index = jnp.stack([src, dst], axis=0)
    edge_attr = attr_out.reshape(Bv * Nv * _K, _D)
    return x, edge_index, edge_attr


# speedup vs baseline: 1.3933x; 1.3560x over previous
import jax
import jax.numpy as jnp
import numpy as np
from jax.experimental import pallas as pl

_B, _N, _K, _D = 8, 2048, 16, 128
_TILE = 512


def _knn_body(lxc, lyc, lxr, lyr, ew, bt, idx_out, attr_out):
    b = pl.program_id(0)
    i = pl.program_id(1)

    xc = lxc[0]
    yc = lyc[0]
    xr = lxr[0]
    yr = lyr[0]

    xcb = xc.astype(jnp.bfloat16).astype(jnp.float32)
    ycb = yc.astype(jnp.bfloat16).astype(jnp.float32)
    xrb = xr.astype(jnp.bfloat16).astype(jnp.float32)
    yrb = yr.astype(jnp.bfloat16).astype(jnp.float32)
    dot = xcb * xrb + ycb * yrb
    sqc = xc * xc + yc * yc
    sqr = xr * xr + yr * yr
    d2 = (sqc + sqr) - 2.0 * dot

    row_g = jax.lax.broadcasted_iota(jnp.int32, (_TILE, _N), 0) + i * _TILE
    col = jax.lax.broadcasted_iota(jnp.int32, (_TILE, _N), 1)
    vals = jnp.where(row_g == col, jnp.float32(1e38), jnp.maximum(d2, 1e-12))

    key = jax.lax.bitcast_convert_type(
        (jax.lax.bitcast_convert_type(vals, jnp.int32)
         & jnp.int32(~0x7FF)) | col, jnp.float32)

    koi = jax.lax.broadcasted_iota(jnp.int32, (_TILE, _K), 1)
    ksel = jnp.zeros((_TILE, _K), jnp.float32)
    imax = jnp.float32(3e38)

    cols = [key[:, c * 128:(c + 1) * 128] for c in range(16)]

    def merge22(a1, a2, b1, b2):
        c1 = jnp.minimum(a1, b1)
        h = jnp.maximum(a1, b1)
        lo = jnp.minimum(a2, b2)
        c4 = jnp.maximum(a2, b2)
        return c1, jnp.minimum(h, lo), jnp.maximum(h, lo), c4

    def merge44_low(a, b):
        t = [jnp.minimum(a[i], b[3 - i]) for i in range(4)]
        x0, x2 = jnp.minimum(t[0], t[2]), jnp.maximum(t[0], t[2])
        x1, x3 = jnp.minimum(t[1], t[3]), jnp.maximum(t[1], t[3])
        return (jnp.minimum(x0, x1), jnp.maximum(x0, x1),
                jnp.minimum(x2, x3), jnp.maximum(x2, x3))

    m_last = None
    for batch in range(_K // 4):
        if m_last is None:
            cf = cols
        else:
            cf = [jnp.where(c > m_last, c, imax) for c in cols]
        s2 = [(jnp.minimum(cf[2 * i], cf[2 * i + 1]),
               jnp.maximum(cf[2 * i], cf[2 * i + 1])) for i in range(8)]
        s4 = [merge22(s2[2 * i][0], s2[2 * i][1],
                      s2[2 * i + 1][0], s2[2 * i + 1][1]) for i in range(4)]
        s4 = [merge44_low(s4[0], s4[1]), merge44_low(s4[2], s4[3])]
        c1, c2, c3, c4 = merge44_low(s4[0], s4[1])
        for t in range(4):
            m = jnp.min(c1, axis=1, keepdims=True)
            eq = c1 == m
            c1 = jnp.where(eq, c2, c1)
            c2 = jnp.where(eq, c3, c2)
            c3 = jnp.where(eq, c4, c3)
            c4 = jnp.where(eq, imax, c4)
            ksel = jnp.where(koi == (4 * batch + t), m, ksel)
            m_last = m

    kseli = jax.lax.bitcast_convert_type(ksel, jnp.int32)
    isel = kseli & jnp.int32(0x7FF)
    dsel = jax.lax.bitcast_convert_type(kseli & jnp.int32(~0x7FF), jnp.float32)

    idx_out[0] = isel + b * _N
    ed = jnp.sqrt(dsel)
    attr = jax.lax.dot_general(
        ed, ew[...],
        dimension_numbers=(((1,), (0,)), ((), ())),
        preferred_element_type=jnp.float32,
        precision=jax.lax.Precision.HIGHEST,
    )
    attr_out[0] = attr + bt[...]


def kernel(locs, init_embeddings, W, b):
    Bv, Nv, _ = locs.shape
    lxc = locs[:, :, 0:1]
    lyc = locs[:, :, 1:2]
    lxr = locs[:, :, 0].reshape(Bv, 1, Nv)
    lyr = locs[:, :, 1].reshape(Bv, 1, Nv)
    Wv = W.reshape(_D)
    ew = jnp.kron(jnp.eye(_K, dtype=jnp.float32), Wv[None, :])
    bt = jnp.tile(b, _K)[None, :]

    grid = (Bv, Nv // _TILE)
    idx_out, attr_out = pl.pallas_call(
        _knn_body,
        grid=grid,
        in_specs=[
            pl.BlockSpec((1, _TILE, 1), lambda b_, i: (b_, i, 0)),
            pl.BlockSpec((1, _TILE, 1), lambda b_, i: (b_, i, 0)),
            pl.BlockSpec((1, 1, Nv), lambda b_, i: (b_, 0, 0)),
            pl.BlockSpec((1, 1, Nv), lambda b_, i: (b_, 0, 0)),
            pl.BlockSpec((_K, _K * _D), lambda b_, i: (0, 0)),
            pl.BlockSpec((1, _K * _D), lambda b_, i: (0, 0)),
        ],
        out_specs=[
            pl.BlockSpec((1, _TILE, _K), lambda b_, i: (b_, i, 0)),
            pl.BlockSpec((1, _TILE, _K * _D), lambda b_, i: (b_, i, 0)),
        ],
        out_shape=[
            jax.ShapeDtypeStruct((Bv, Nv, _K), jnp.int32),
            jax.ShapeDtypeStruct((Bv, Nv, _K * _D), jnp.float32),
        ],
    )(lxc, lyc, lxr, lyr, ew, bt)

    x = init_embeddings.reshape(Bv * Nv, _D)
    src = jnp.broadcast_to(
        jnp.arange(Bv * Nv, dtype=jnp.int32)[:, None], (Bv * Nv, _K)
    ).reshape(-1)
    dst = idx_out.reshape(-1)
    edge_index = jnp.stack([src, dst], axis=0)
    edge_attr = attr_out.reshape(Bv * Nv * _K, _D)
    return x, edge_index, edge_attr


# speedup vs baseline: 1.7731x; 1.2725x over previous
import jax
import jax.numpy as jnp
import numpy as np
from jax.experimental import pallas as pl

_B, _N, _K, _D = 8, 2048, 16, 128
_TILE = 512


def _knn_body(lxc, lyc, lxr, lyr, ew, bt, idx_out, attr_out):
    b = pl.program_id(0)
    i = pl.program_id(1)

    xc = lxc[0]
    yc = lyc[0]
    xr = lxr[0]
    yr = lyr[0]

    xcb = xc.astype(jnp.bfloat16).astype(jnp.float32)
    ycb = yc.astype(jnp.bfloat16).astype(jnp.float32)
    xrb = xr.astype(jnp.bfloat16).astype(jnp.float32)
    yrb = yr.astype(jnp.bfloat16).astype(jnp.float32)
    dot = xcb * xrb + ycb * yrb
    sqc = xc * xc + yc * yc
    sqr = xr * xr + yr * yr
    d2 = (sqc + sqr) - 2.0 * dot

    row_g = jax.lax.broadcasted_iota(jnp.int32, (_TILE, _N), 0) + i * _TILE
    col = jax.lax.broadcasted_iota(jnp.int32, (_TILE, _N), 1)
    vals = jnp.where(row_g == col, jnp.float32(1e38), jnp.maximum(d2, 1e-12))

    key = jax.lax.bitcast_convert_type(
        (jax.lax.bitcast_convert_type(vals, jnp.int32)
         & jnp.int32(~0x7FF)) | col, jnp.float32)

    koi = jax.lax.broadcasted_iota(jnp.int32, (_TILE, _K), 1)
    ksel = jnp.zeros((_TILE, _K), jnp.float32)
    imax = jnp.float32(3e38)

    cols = [key[:, c * 128:(c + 1) * 128] for c in range(16)]

    def merge22(a1, a2, b1, b2):
        c1 = jnp.minimum(a1, b1)
        h = jnp.maximum(a1, b1)
        lo = jnp.minimum(a2, b2)
        c4 = jnp.maximum(a2, b2)
        return c1, jnp.minimum(h, lo), jnp.maximum(h, lo), c4

    _BITONIC8 = [(0, 4), (1, 5), (2, 6), (3, 7), (0, 2), (1, 3), (4, 6),
                 (5, 7), (0, 1), (2, 3), (4, 5), (6, 7)]

    def merge44_all(a, b):
        x = [a[0], a[1], a[2], a[3], b[3], b[2], b[1], b[0]]
        for lo, hi in _BITONIC8:
            x[lo], x[hi] = jnp.minimum(x[lo], x[hi]), jnp.maximum(x[lo], x[hi])
        return x

    def merge88_low(a, b):
        x = [jnp.minimum(a[i], b[7 - i]) for i in range(8)]
        for lo, hi in _BITONIC8:
            x[lo], x[hi] = jnp.minimum(x[lo], x[hi]), jnp.maximum(x[lo], x[hi])
        return x

    m_last = None
    for batch in range(_K // 8):
        if m_last is None:
            cf = cols
        else:
            cf = [jnp.where(c > m_last, c, imax) for c in cols]
        s2 = [(jnp.minimum(cf[2 * i], cf[2 * i + 1]),
               jnp.maximum(cf[2 * i], cf[2 * i + 1])) for i in range(8)]
        s4 = [merge22(s2[2 * i][0], s2[2 * i][1],
                      s2[2 * i + 1][0], s2[2 * i + 1][1]) for i in range(4)]
        s8 = [merge44_all(s4[0], s4[1]), merge44_all(s4[2], s4[3])]
        c = merge88_low(s8[0], s8[1])
        for t in range(8):
            m = jnp.min(c[0], axis=1, keepdims=True)
            eq = c[0] == m
            for k in range(7):
                c[k] = jnp.where(eq, c[k + 1], c[k])
            c[7] = jnp.where(eq, imax, c[7])
            ksel = jnp.where(koi == (8 * batch + t), m, ksel)
            m_last = m

    kseli = jax.lax.bitcast_convert_type(ksel, jnp.int32)
    isel = kseli & jnp.int32(0x7FF)
    dsel = jax.lax.bitcast_convert_type(kseli & jnp.int32(~0x7FF), jnp.float32)

    idx_out[0] = isel + b * _N
    ed = jnp.sqrt(dsel)
    attr = jax.lax.dot_general(
        ed, ew[...],
        dimension_numbers=(((1,), (0,)), ((), ())),
        preferred_element_type=jnp.float32,
        precision=jax.lax.Precision.DEFAULT,
    )
    attr_out[0] = attr + bt[...]


def kernel(locs, init_embeddings, W, b):
    Bv, Nv, _ = locs.shape
    lxc = locs[:, :, 0:1]
    lyc = locs[:, :, 1:2]
    lxr = locs[:, :, 0].reshape(Bv, 1, Nv)
    lyr = locs[:, :, 1].reshape(Bv, 1, Nv)
    Wv = W.reshape(_D)
    ew = jnp.kron(jnp.eye(_K, dtype=jnp.float32), Wv[None, :])
    bt = jnp.tile(b, _K)[None, :]

    grid = (Bv, Nv // _TILE)
    idx_out, attr_out = pl.pallas_call(
        _knn_body,
        grid=grid,
        in_specs=[
            pl.BlockSpec((1, _TILE, 1), lambda b_, i: (b_, i, 0)),
            pl.BlockSpec((1, _TILE, 1), lambda b_, i: (b_, i, 0)),
            pl.BlockSpec((1, 1, Nv), lambda b_, i: (b_, 0, 0)),
            pl.BlockSpec((1, 1, Nv), lambda b_, i: (b_, 0, 0)),
            pl.BlockSpec((_K, _K * _D), lambda b_, i: (0, 0)),
            pl.BlockSpec((1, _K * _D), lambda b_, i: (0, 0)),
        ],
        out_specs=[
            pl.BlockSpec((1, _TILE, _K), lambda b_, i: (b_, i, 0)),
            pl.BlockSpec((1, _TILE, _K * _D), lambda b_, i: (b_, i, 0)),
        ],
        out_shape=[
            jax.ShapeDtypeStruct((Bv, Nv, _K), jnp.int32),
            jax.ShapeDtypeStruct((Bv, Nv, _K * _D), jnp.float32),
        ],
    )(lxc, lyc, lxr, lyr, ew, bt)

    x = init_embeddings.reshape(Bv * Nv, _D)
    src = jnp.broadcast_to(
        jnp.arange(Bv * Nv, dtype=jnp.int32)[:, None], (Bv * Nv, _K)
    ).reshape(-1)
    dst = idx_out.reshape(-1)
    edge_index = jnp.stack([src, dst], axis=0)
    edge_attr = attr_out.reshape(Bv * Nv * _K, _D)
    return x, edge_index, edge_attr
